# Initial kernel scaffold; baseline (speedup 1.0000x reference)
import jax
import jax.numpy as jnp
from jax.experimental import pallas as pl


def kernel(x, row, alpha, W1, b1, W2, b2, ln_g, ln_b, Wa, ba):
    raise NotImplementedError("write your pallas kernel here")


# speedup vs baseline: 3.0177x; 3.0177x over previous
import functools

import jax
import jax.numpy as jnp
from jax import lax
from jax.experimental import pallas as pl
from jax.experimental.pallas import tpu as pltpu
from jax.experimental.pallas import tpu_sc as plsc

N = 160000
D = 256
HP = 16
NSEG = 10000
EPS = 1e-5

TB = 640
TB2 = 4000

NW = 32
CH = N // NW
CW = 125
NCH = CH // CW
NSEGP = 10240
SEG_STRIPE = NSEGP // 16


def _tc1_body(x_ref, w1_ref, w2_ref, wa_ref, b1_ref, b2_ref, g_ref, be_ref,
              ba_ref, mask_ref, o_ref):
    xb = x_ref[...].astype(jnp.bfloat16)
    h = jnp.dot(xb, w1_ref[...], preferred_element_type=jnp.float32)
    h = jnp.maximum(h + b1_ref[...], 0.0)
    h = jnp.dot(h.astype(jnp.bfloat16), w2_ref[...],
                preferred_element_type=jnp.float32)
    h = jnp.maximum(h + b2_ref[...], 0.0)
    mu = jnp.mean(h, axis=-1, keepdims=True)
    var = jnp.mean((h - mu) ** 2, axis=-1, keepdims=True)
    hn = (h - mu) / jnp.sqrt(var + EPS) * g_ref[...] + be_ref[...]
    raw = jnp.dot(hn.astype(jnp.bfloat16), wa_ref[...],
                  preferred_element_type=jnp.float32) + ba_ref[...]
    s = jax.nn.sigmoid(raw)
    o_ref[...] = jnp.exp(s) * mask_ref[...]


def _tc1(x, w1t, w2t, wat, b1r, b2r, gr, ber, bar, maskr):
    grid = (N // TB,)
    full = lambda shape: pl.BlockSpec(shape, lambda i: (0, 0))
    return pl.pallas_call(
        _tc1_body,
        grid=grid,
        in_specs=[
            pl.BlockSpec((TB, D), lambda i: (i, 0)),
            full((D, D)), full((D, D)), full((D, HP)),
            full((1, D)), full((1, D)), full((1, D)), full((1, D)),
            full((1, HP)), full((1, HP)),
        ],
        out_specs=pl.BlockSpec((TB, HP), lambda i: (i, 0)),
        out_shape=jax.ShapeDtypeStruct((N, HP), jnp.float32),
    )(x, w1t, w2t, wat, b1r, b2r, gr, ber, bar, maskr)


def _sc_mesh():
    return plsc.VectorSubcoreMesh(core_axis_name="c", subcore_axis_name="s")


@jax.jit
def _sc_k1(e, row3, zeros):
    @functools.partial(
        pl.kernel,
        out_type=jax.ShapeDtypeStruct((2, NSEGP, HP), jnp.float32),
        mesh=_sc_mesh(),
        compiler_params=pltpu.CompilerParams(use_tc_tiling_on_sc=False),
        scratch_types=[
            pltpu.VMEM((CH, HP), jnp.float32),
            pltpu.VMEM((NCH, CW), jnp.int32),
            pltpu.VMEM_SHARED((NSEGP, HP), jnp.float32),
            pltpu.SemaphoreType.DMA,
        ],
    )
    def k(e_hbm, row_hbm, z_hbm, p_hbm, e_v, row_v, acc_sh, sem):
        c = lax.axis_index("c")
        s = lax.axis_index("s")
        wid = s * 2 + c

        @pl.when(s == 0)
        def _():
            pltpu.sync_copy(z_hbm, acc_sh)

        pltpu.async_copy(e_hbm.at[pl.ds(wid * CH, CH)], e_v, sem).wait()
        pltpu.async_copy(row_hbm.at[wid], row_v, sem).wait()
        plsc.subcore_barrier()

        @pl.loop(0, NCH)
        def _(j):
            pltpu.sync_copy(e_v.at[pl.ds(j * CW, CW)],
                            acc_sh.at[row_v.at[j]], add=True)

        plsc.subcore_barrier()
        pltpu.sync_copy(acc_sh.at[pl.ds(s * SEG_STRIPE, SEG_STRIPE)],
                        p_hbm.at[c].at[pl.ds(s * SEG_STRIPE, SEG_STRIPE)])

    return k(e, row3, zeros)


@jax.jit
def _sc_k2(recip, row3):
    @functools.partial(
        pl.kernel,
        out_type=jax.ShapeDtypeStruct((N, HP), jnp.float32),
        mesh=_sc_mesh(),
        compiler_params=pltpu.CompilerParams(use_tc_tiling_on_sc=False),
        scratch_types=[
            pltpu.VMEM((CH, HP), jnp.float32),
            pltpu.VMEM((NCH, CW), jnp.int32),
            pltpu.SemaphoreType.DMA,
        ],
    )
    def k(recip_hbm, row_hbm, r_hbm, g_v, row_v, sem):
        c = lax.axis_index("c")
        s = lax.axis_index("s")
        wid = s * 2 + c
        pltpu.async_copy(row_hbm.at[wid], row_v, sem).wait()

        @pl.loop(0, NCH)
        def _(j):
            pltpu.sync_copy(recip_hbm.at[row_v.at[j]],
                            g_v.at[pl.ds(j * CW, CW)])

        pltpu.sync_copy(g_v, r_hbm.at[pl.ds(wid * CH, CH)])

    return k(recip, row3)


def _tcmid_body(p_ref, o_ref):
    acc = p_ref[0] + p_ref[1]
    o_ref[...] = jnp.where(acc > 0.0, 1.0 / acc, 0.0)


def _tcmid(partials):
    return pl.pallas_call(
        _tcmid_body,
        in_specs=[pl.BlockSpec((2, NSEGP, HP), lambda: (0, 0, 0))],
        out_specs=pl.BlockSpec((NSEGP, HP), lambda: (0, 0)),
        out_shape=jax.ShapeDtypeStruct((NSEGP, HP), jnp.float32),
    )(partials)


def _tc2_body(e_ref, r_ref, o_ref):
    prod = e_ref[...] * r_ref[...]
    o_ref[...] = 0.25 * jnp.sum(prod, axis=1, keepdims=True)


def _tc2(e, r):
    grid = (N // TB2,)
    return pl.pallas_call(
        _tc2_body,
        grid=grid,
        in_specs=[pl.BlockSpec((TB2, HP), lambda i: (i, 0)),
                  pl.BlockSpec((TB2, HP), lambda i: (i, 0))],
        out_specs=pl.BlockSpec((TB2, 1), lambda i: (i, 0)),
        out_shape=jax.ShapeDtypeStruct((N, 1), jnp.float32),
    )(e, r)


def kernel(x, row, alpha, W1, b1, W2, b2, ln_g, ln_b, Wa, ba):
    b1_eff = (b1 + alpha[0, 0] * W1[:, D]).reshape(1, D)
    w1t = W1[:, :D].T.astype(jnp.bfloat16)
    w2t = W2.T.astype(jnp.bfloat16)
    wa_pad = jnp.zeros((HP, D), jnp.float32).at[:4].set(Wa)
    wat = wa_pad.T.astype(jnp.bfloat16)
    ba_pad = jnp.zeros((1, HP), jnp.float32).at[0, :4].set(ba)
    mask = jnp.zeros((1, HP), jnp.float32).at[0, :4].set(1.0)
    gr = ln_g.reshape(1, D)
    ber = ln_b.reshape(1, D)
    row3 = row.reshape(NW, NCH, CW)
    zeros = jnp.zeros((NSEGP, HP), jnp.float32)

    e = _tc1(x, w1t, w2t, wat, b1_eff, b2.reshape(1, D), gr, ber, ba_pad,
             mask)
    partials = _sc_k1(e, row3, zeros)
    recip = _tcmid(partials)
    r = _sc_k2(recip, row3)
    return _tc2(e, r)


# speedup vs baseline: 3.8126x; 1.2634x over previous
import functools

import jax
import jax.numpy as jnp
from jax import lax
from jax.experimental import pallas as pl
from jax.experimental.pallas import tpu as pltpu
from jax.experimental.pallas import tpu_sc as plsc

N = 160000
D = 256
HP = 16
NSEG = 10000
EPS = 1e-5

TB = 1600
TB2 = 4000

NW = 32
CH = N // NW
CW = 125
NCH = CH // CW
NSEGP = 10240
SEG_STRIPE = NSEGP // 16


def _tc1_body(x_ref, w1_ref, w2_ref, wa_ref, b1_ref, b2_ref, swa_ref, bb_ref,
              mask_ref, o_ref):
    xb = x_ref[...].astype(jnp.bfloat16)
    h = jnp.dot(xb, w1_ref[...], preferred_element_type=jnp.float32)
    h = jnp.maximum(h + b1_ref[...], 0.0)
    h = jnp.dot(h.astype(jnp.bfloat16), w2_ref[...],
                preferred_element_type=jnp.float32)
    h = jnp.maximum(h + b2_ref[...], 0.0)
    s1 = jnp.sum(h, axis=-1, keepdims=True)
    s2 = jnp.sum(h * h, axis=-1, keepdims=True)
    mu = s1 * (1.0 / D)
    var = s2 * (1.0 / D) - mu * mu
    inv = jax.lax.rsqrt(var + EPS)
    t = jnp.dot(h.astype(jnp.bfloat16), wa_ref[...],
                preferred_element_type=jnp.float32)
    raw = inv * (t - mu * swa_ref[...]) + bb_ref[...]
    s = jax.nn.sigmoid(raw)
    o_ref[...] = jnp.exp(s) * mask_ref[...]


def _tc1(x, w1t, w2t, wat, b1r, b2r, swar, bbr, maskr):
    grid = (N // TB,)
    full = lambda shape: pl.BlockSpec(shape, lambda i: (0, 0))
    return pl.pallas_call(
        _tc1_body,
        grid=grid,
        in_specs=[
            pl.BlockSpec((TB, D), lambda i: (i, 0)),
            full((D, D)), full((D, D)), full((D, HP)),
            full((1, D)), full((1, D)),
            full((1, HP)), full((1, HP)), full((1, HP)),
        ],
        out_specs=pl.BlockSpec((TB, HP), lambda i: (i, 0)),
        out_shape=jax.ShapeDtypeStruct((N, HP), jnp.float32),
    )(x, w1t, w2t, wat, b1r, b2r, swar, bbr, maskr)


def _sc_mesh():
    return plsc.VectorSubcoreMesh(core_axis_name="c", subcore_axis_name="s")


@jax.jit
def _sc_k1(e, row3, zeros):
    @functools.partial(
        pl.kernel,
        out_type=jax.ShapeDtypeStruct((2, NSEGP, HP), jnp.float32),
        mesh=_sc_mesh(),
        compiler_params=pltpu.CompilerParams(use_tc_tiling_on_sc=False),
        scratch_types=[
            pltpu.VMEM((CH, HP), jnp.float32),
            pltpu.VMEM((NCH, CW), jnp.int32),
            pltpu.VMEM_SHARED((NSEGP, HP), jnp.float32),
            pltpu.SemaphoreType.DMA,
        ],
    )
    def k(e_hbm, row_hbm, z_hbm, p_hbm, e_v, row_v, acc_sh, sem):
        c = lax.axis_index("c")
        s = lax.axis_index("s")
        wid = s * 2 + c

        @pl.when(s == 0)
        def _():
            pltpu.sync_copy(z_hbm, acc_sh)

        pltpu.async_copy(e_hbm.at[pl.ds(wid * CH, CH)], e_v, sem).wait()
        pltpu.async_copy(row_hbm.at[wid], row_v, sem).wait()
        plsc.subcore_barrier()

        @pl.loop(0, NCH)
        def _(j):
            pltpu.sync_copy(e_v.at[pl.ds(j * CW, CW)],
                            acc_sh.at[row_v.at[j]], add=True)

        plsc.subcore_barrier()
        pltpu.sync_copy(acc_sh.at[pl.ds(s * SEG_STRIPE, SEG_STRIPE)],
                        p_hbm.at[c].at[pl.ds(s * SEG_STRIPE, SEG_STRIPE)])

    return k(e, row3, zeros)


@jax.jit
def _sc_k2(recip, row3):
    @functools.partial(
        pl.kernel,
        out_type=jax.ShapeDtypeStruct((N, HP), jnp.float32),
        mesh=_sc_mesh(),
        compiler_params=pltpu.CompilerParams(use_tc_tiling_on_sc=False),
        scratch_types=[
            pltpu.VMEM((CH, HP), jnp.float32),
            pltpu.VMEM((NCH, CW), jnp.int32),
            pltpu.SemaphoreType.DMA,
        ],
    )
    def k(recip_hbm, row_hbm, r_hbm, g_v, row_v, sem):
        c = lax.axis_index("c")
        s = lax.axis_index("s")
        wid = s * 2 + c
        pltpu.async_copy(row_hbm.at[wid], row_v, sem).wait()

        @pl.loop(0, NCH)
        def _(j):
            pltpu.sync_copy(recip_hbm.at[row_v.at[j]],
                            g_v.at[pl.ds(j * CW, CW)])

        pltpu.sync_copy(g_v, r_hbm.at[pl.ds(wid * CH, CH)])

    return k(recip, row3)


def _tcmid_body(p_ref, o_ref):
    acc = p_ref[0] + p_ref[1]
    o_ref[...] = jnp.where(acc > 0.0, 1.0 / acc, 0.0)


def _tcmid(partials):
    return pl.pallas_call(
        _tcmid_body,
        in_specs=[pl.BlockSpec((2, NSEGP, HP), lambda: (0, 0, 0))],
        out_specs=pl.BlockSpec((NSEGP, HP), lambda: (0, 0)),
        out_shape=jax.ShapeDtypeStruct((NSEGP, HP), jnp.float32),
    )(partials)


def _tc2_body(e_ref, r_ref, o_ref):
    prod = e_ref[...] * r_ref[...]
    o_ref[...] = 0.25 * jnp.sum(prod, axis=1, keepdims=True)


def _tc2(e, r):
    grid = (N // TB2,)
    return pl.pallas_call(
        _tc2_body,
        grid=grid,
        in_specs=[pl.BlockSpec((TB2, HP), lambda i: (i, 0)),
                  pl.BlockSpec((TB2, HP), lambda i: (i, 0))],
        out_specs=pl.BlockSpec((TB2, 1), lambda i: (i, 0)),
        out_shape=jax.ShapeDtypeStruct((N, 1), jnp.float32),
    )(e, r)


def kernel(x, row, alpha, W1, b1, W2, b2, ln_g, ln_b, Wa, ba):
    b1_eff = (b1 + alpha[0, 0] * W1[:, D]).reshape(1, D)
    w1t = W1[:, :D].T.astype(jnp.bfloat16)
    w2t = W2.T.astype(jnp.bfloat16)
    wa_pad = jnp.zeros((HP, D), jnp.float32).at[:4].set(Wa)
    wa_g = wa_pad * ln_g[None, :]
    wat = wa_g.T.astype(jnp.bfloat16)
    swa = jnp.sum(wa_g, axis=1).reshape(1, HP)
    bb = (jnp.zeros((HP,), jnp.float32).at[:4].set(ba)
          + wa_pad @ ln_b).reshape(1, HP)
    mask = jnp.zeros((1, HP), jnp.float32).at[0, :4].set(1.0)
    row3 = row.reshape(NW, NCH, CW)
    zeros = jnp.zeros((NSEGP, HP), jnp.float32)

    e = _tc1(x, w1t, w2t, wat, b1_eff, b2.reshape(1, D), swa, bb, mask)
    partials = _sc_k1(e, row3, zeros)
    recip = _tcmid(partials)
    r = _sc_k2(recip, row3)
    return _tc2(e, r)


# speedup vs baseline: 5.1528x; 1.3515x over previous
import functools

import jax
import jax.numpy as jnp
from jax import lax
from jax.experimental import pallas as pl
from jax.experimental.pallas import tpu as pltpu
from jax.experimental.pallas import tpu_sc as plsc

N = 160000
D = 256
HP = 16
NSEG = 10000
EPS = 1e-5

TB = 1600
TB2L = 2048
LIN = N * HP // 128
OLIN = N // 128

NW = 32
CH = N // NW
CW = 125
NCH = CH // CW
NSEGP = 10240
SEG_STRIPE = NSEGP // 16


def _tc1_body(x_ref, w1_ref, w2_ref, wa_ref, b1_ref, b2_ref, swa_ref, bb_ref,
              mask_ref, o_ref):
    xb = x_ref[...].astype(jnp.bfloat16)
    h = jnp.dot(xb, w1_ref[...], preferred_element_type=jnp.float32)
    h = jnp.maximum(h + b1_ref[...], 0.0)
    h = jnp.dot(h.astype(jnp.bfloat16), w2_ref[...],
                preferred_element_type=jnp.float32)
    h = jnp.maximum(h + b2_ref[...], 0.0)
    s1 = jnp.sum(h, axis=-1, keepdims=True)
    s2 = jnp.sum(h * h, axis=-1, keepdims=True)
    mu = s1 * (1.0 / D)
    var = s2 * (1.0 / D) - mu * mu
    inv = jax.lax.rsqrt(var + EPS)
    t = jnp.dot(h.astype(jnp.bfloat16), wa_ref[...],
                preferred_element_type=jnp.float32)
    raw = inv * (t - mu * swa_ref[...]) + bb_ref[...]
    s = jax.nn.sigmoid(raw)
    o_ref[...] = jnp.exp(s) * mask_ref[...]


def _tc1(x, w1t, w2t, wat, b1r, b2r, swar, bbr, maskr):
    grid = (N // TB,)
    full = lambda shape: pl.BlockSpec(shape, lambda i: (0, 0))
    return pl.pallas_call(
        _tc1_body,
        grid=grid,
        in_specs=[
            pl.BlockSpec((TB, D), lambda i: (i, 0)),
            full((D, D)), full((D, D)), full((D, HP)),
            full((1, D)), full((1, D)),
            full((1, HP)), full((1, HP)), full((1, HP)),
        ],
        out_specs=pl.BlockSpec((TB, HP), lambda i: (i, 0)),
        out_shape=jax.ShapeDtypeStruct((N, HP), jnp.float32),
    )(x, w1t, w2t, wat, b1r, b2r, swar, bbr, maskr)


def _sc_mesh():
    return plsc.VectorSubcoreMesh(core_axis_name="c", subcore_axis_name="s")


@jax.jit
def _sc_k1(e, row3, zeros):
    @functools.partial(
        pl.kernel,
        out_type=jax.ShapeDtypeStruct((2, NSEGP, HP), jnp.float32),
        mesh=_sc_mesh(),
        compiler_params=pltpu.CompilerParams(use_tc_tiling_on_sc=False),
        scratch_types=[
            pltpu.VMEM((CH, HP), jnp.float32),
            pltpu.VMEM((NCH, CW), jnp.int32),
            pltpu.VMEM_SHARED((NSEGP, HP), jnp.float32),
            pltpu.SemaphoreType.DMA,
        ],
    )
    def k(e_hbm, row_hbm, z_hbm, p_hbm, e_v, row_v, acc_sh, sem):
        c = lax.axis_index("c")
        s = lax.axis_index("s")
        wid = s * 2 + c

        @pl.when(s == 0)
        def _():
            pltpu.sync_copy(z_hbm, acc_sh)

        pltpu.async_copy(e_hbm.at[pl.ds(wid * CH, CH)], e_v, sem).wait()
        pltpu.async_copy(row_hbm.at[wid], row_v, sem).wait()
        plsc.subcore_barrier()

        @pl.loop(0, NCH)
        def _(j):
            pltpu.sync_copy(e_v.at[pl.ds(j * CW, CW)],
                            acc_sh.at[row_v.at[j]], add=True)

        plsc.subcore_barrier()
        pltpu.sync_copy(acc_sh.at[pl.ds(s * SEG_STRIPE, SEG_STRIPE)],
                        p_hbm.at[c].at[pl.ds(s * SEG_STRIPE, SEG_STRIPE)])

    return k(e, row3, zeros)


@jax.jit
def _sc_k2(recip, row3):
    @functools.partial(
        pl.kernel,
        out_type=jax.ShapeDtypeStruct((N, HP), jnp.float32),
        mesh=_sc_mesh(),
        compiler_params=pltpu.CompilerParams(use_tc_tiling_on_sc=False),
        scratch_types=[
            pltpu.VMEM((CH, HP), jnp.float32),
            pltpu.VMEM((NCH, CW), jnp.int32),
            pltpu.SemaphoreType.DMA,
        ],
    )
    def k(recip_hbm, row_hbm, r_hbm, g_v, row_v, sem):
        c = lax.axis_index("c")
        s = lax.axis_index("s")
        wid = s * 2 + c
        pltpu.async_copy(row_hbm.at[wid], row_v, sem).wait()

        @pl.loop(0, NCH)
        def _(j):
            pltpu.sync_copy(recip_hbm.at[row_v.at[j]],
                            g_v.at[pl.ds(j * CW, CW)])

        pltpu.sync_copy(g_v, r_hbm.at[pl.ds(wid * CH, CH)])

    return k(recip, row3)


def _tcmid_body(p_ref, o_ref):
    acc = p_ref[0] + p_ref[1]
    o_ref[...] = jnp.where(acc > 0.0, 1.0 / acc, 0.0)


def _tcmid(partials):
    return pl.pallas_call(
        _tcmid_body,
        in_specs=[pl.BlockSpec((2, NSEGP, HP), lambda: (0, 0, 0))],
        out_specs=pl.BlockSpec((NSEGP, HP), lambda: (0, 0)),
        out_shape=jax.ShapeDtypeStruct((NSEGP, HP), jnp.float32),
    )(partials)


def _tc2_body(e_ref, r_ref, g_ref, o_ref):
    prod = e_ref[...] * r_ref[...]
    sums = jnp.dot(prod, g_ref[...], preferred_element_type=jnp.float32)
    o_ref[...] = 0.25 * sums


def _tc2(e_lin, r_lin, g):
    grid = ((LIN + TB2L - 1) // TB2L,)
    return pl.pallas_call(
        _tc2_body,
        grid=grid,
        in_specs=[pl.BlockSpec((TB2L, 128), lambda i: (i, 0)),
                  pl.BlockSpec((TB2L, 128), lambda i: (i, 0)),
                  pl.BlockSpec((128, 8), lambda i: (0, 0))],
        out_specs=pl.BlockSpec((TB2L, 8), lambda i: (i, 0)),
        out_shape=jax.ShapeDtypeStruct((LIN, 8), jnp.float32),
    )(e_lin, r_lin, g)


def kernel(x, row, alpha, W1, b1, W2, b2, ln_g, ln_b, Wa, ba):
    b1_eff = (b1 + alpha[0, 0] * W1[:, D]).reshape(1, D)
    w1t = W1[:, :D].T.astype(jnp.bfloat16)
    w2t = W2.T.astype(jnp.bfloat16)
    wa_pad = jnp.zeros((HP, D), jnp.float32).at[:4].set(Wa)
    wa_g = wa_pad * ln_g[None, :]
    wat = wa_g.T.astype(jnp.bfloat16)
    swa = jnp.sum(wa_g, axis=1).reshape(1, HP)
    bb = (jnp.zeros((HP,), jnp.float32).at[:4].set(ba)
          + wa_pad @ ln_b).reshape(1, HP)
    mask = jnp.zeros((1, HP), jnp.float32).at[0, :4].set(1.0)
    row3 = row.reshape(NW, NCH, CW)
    zeros = jnp.zeros((NSEGP, HP), jnp.float32)

    g = (jnp.arange(128)[:, None] // HP ==
         jnp.arange(8)[None, :]).astype(jnp.float32)

    e16 = _tc1(x, w1t, w2t, wat, b1_eff, b2.reshape(1, D), swa, bb, mask)
    partials = _sc_k1(e16, row3, zeros)
    recip = _tcmid(partials)
    r16 = _sc_k2(recip, row3)
    out_lin = _tc2(e16.reshape(LIN, 128), r16.reshape(LIN, 128), g)
    return out_lin.reshape(N, 1)


# speedup vs baseline: 6.0331x; 1.1708x over previous
import functools

import jax
import jax.numpy as jnp
from jax import lax
from jax.experimental import pallas as pl
from jax.experimental.pallas import tpu as pltpu
from jax.experimental.pallas import tpu_sc as plsc

N = 160000
D = 256
HP = 16
NSEG = 10000
EPS = 1e-5

TB = 1600
TB2L = 2048
LIN = N * HP // 128
OLIN = N // 128

NW = 32
CH = N // NW
CW = 125
NCH = CH // CW
NSEGP = 10240
SEG_STRIPE = NSEGP // 16


def _tc1_body(x_ref, w1_ref, w2_ref, wa_ref, b1_ref, b2_ref, swa_ref, bb_ref,
              mask_ref, o_ref):
    xb = x_ref[...].astype(jnp.bfloat16)
    h = jnp.dot(xb, w1_ref[...], preferred_element_type=jnp.float32)
    h = jnp.maximum(h + b1_ref[...], 0.0)
    h = jnp.dot(h.astype(jnp.bfloat16), w2_ref[...],
                preferred_element_type=jnp.float32)
    h = jnp.maximum(h + b2_ref[...], 0.0)
    s1 = jnp.sum(h, axis=-1, keepdims=True)
    s2 = jnp.sum(h * h, axis=-1, keepdims=True)
    mu = s1 * (1.0 / D)
    var = s2 * (1.0 / D) - mu * mu
    inv = jax.lax.rsqrt(var + EPS)
    t = jnp.dot(h.astype(jnp.bfloat16), wa_ref[...],
                preferred_element_type=jnp.float32)
    raw = inv * (t - mu * swa_ref[...]) + bb_ref[...]
    s = jax.nn.sigmoid(raw)
    e = jnp.exp(s) * mask_ref[...]
    e3 = e.reshape(TB // 8, 8, HP)
    o_ref[...] = jnp.concatenate([e3[:, u, :] for u in range(8)], axis=1)


def _tc1(x, w1t, w2t, wat, b1r, b2r, swar, bbr, maskr):
    grid = (N // TB,)
    full = lambda shape: pl.BlockSpec(shape, lambda i: (0, 0))
    return pl.pallas_call(
        _tc1_body,
        grid=grid,
        in_specs=[
            pl.BlockSpec((TB, D), lambda i: (i, 0)),
            full((D, D)), full((D, D)), full((D, HP)),
            full((1, D)), full((1, D)),
            full((1, HP)), full((1, HP)), full((1, HP)),
        ],
        out_specs=pl.BlockSpec((TB * HP // 128, 128), lambda i: (i, 0)),
        out_shape=jax.ShapeDtypeStruct((LIN, 128), jnp.float32),
    )(x, w1t, w2t, wat, b1r, b2r, swar, bbr, maskr)


def _sc_mesh():
    return plsc.VectorSubcoreMesh(core_axis_name="c", subcore_axis_name="s")


@jax.jit
def _sc_k1(e, row3, zeros):
    @functools.partial(
        pl.kernel,
        out_type=jax.ShapeDtypeStruct((2, NSEGP, HP), jnp.float32),
        mesh=_sc_mesh(),
        compiler_params=pltpu.CompilerParams(use_tc_tiling_on_sc=False),
        scratch_types=[
            pltpu.VMEM((CH, HP), jnp.float32),
            pltpu.VMEM((NCH, CW), jnp.int32),
            pltpu.VMEM_SHARED((NSEGP, HP), jnp.float32),
            pltpu.SemaphoreType.DMA,
        ],
    )
    def k(e_hbm, row_hbm, z_hbm, p_hbm, e_v, row_v, acc_sh, sem):
        c = lax.axis_index("c")
        s = lax.axis_index("s")
        wid = s * 2 + c

        @pl.when(s == 0)
        def _():
            pltpu.sync_copy(z_hbm, acc_sh)

        pltpu.async_copy(e_hbm.at[pl.ds(wid * CH, CH)], e_v, sem).wait()
        pltpu.async_copy(row_hbm.at[wid], row_v, sem).wait()
        plsc.subcore_barrier()

        @pl.loop(0, NCH)
        def _(j):
            pltpu.sync_copy(e_v.at[pl.ds(j * CW, CW)],
                            acc_sh.at[row_v.at[j]], add=True)

        plsc.subcore_barrier()
        pltpu.sync_copy(acc_sh.at[pl.ds(s * SEG_STRIPE, SEG_STRIPE)],
                        p_hbm.at[c].at[pl.ds(s * SEG_STRIPE, SEG_STRIPE)])

    return k(e, row3, zeros)


@jax.jit
def _sc_k2(recip, row3):
    @functools.partial(
        pl.kernel,
        out_type=jax.ShapeDtypeStruct((N, HP), jnp.float32),
        mesh=_sc_mesh(),
        compiler_params=pltpu.CompilerParams(use_tc_tiling_on_sc=False),
        scratch_types=[
            pltpu.VMEM((CH, HP), jnp.float32),
            pltpu.VMEM((NCH, CW), jnp.int32),
            pltpu.SemaphoreType.DMA,
        ],
    )
    def k(recip_hbm, row_hbm, r_hbm, g_v, row_v, sem):
        c = lax.axis_index("c")
        s = lax.axis_index("s")
        wid = s * 2 + c
        pltpu.async_copy(row_hbm.at[wid], row_v, sem).wait()

        @pl.loop(0, NCH)
        def _(j):
            pltpu.sync_copy(recip_hbm.at[row_v.at[j]],
                            g_v.at[pl.ds(j * CW, CW)])

        pltpu.sync_copy(g_v, r_hbm.at[pl.ds(wid * CH, CH)])

    return k(recip, row3)


def _tcmid_body(p_ref, o_ref):
    acc = p_ref[0] + p_ref[1]
    o_ref[...] = jnp.where(acc > 0.0, 1.0 / acc, 0.0)


def _tcmid(partials):
    return pl.pallas_call(
        _tcmid_body,
        in_specs=[pl.BlockSpec((2, NSEGP, HP), lambda: (0, 0, 0))],
        out_specs=pl.BlockSpec((NSEGP, HP), lambda: (0, 0)),
        out_shape=jax.ShapeDtypeStruct((NSEGP, HP), jnp.float32),
    )(partials)


def _tc2_body(e_ref, r_ref, g_ref, o_ref):
    prod = e_ref[...] * r_ref[...]
    sums = 0.25 * jnp.dot(prod, g_ref[...], preferred_element_type=jnp.float32)
    s3 = sums.reshape(TB2L // 16, 16, 8)
    o_ref[...] = jnp.concatenate([s3[:, u, :] for u in range(16)], axis=1)


def _tc2(e_lin, r_lin, g):
    grid = ((LIN + TB2L - 1) // TB2L,)
    return pl.pallas_call(
        _tc2_body,
        grid=grid,
        in_specs=[pl.BlockSpec((TB2L, 128), lambda i: (i, 0)),
                  pl.BlockSpec((TB2L, 128), lambda i: (i, 0)),
                  pl.BlockSpec((128, 8), lambda i: (0, 0))],
        out_specs=pl.BlockSpec((TB2L // 16, 128), lambda i: (i, 0)),
        out_shape=jax.ShapeDtypeStruct((OLIN, 128), jnp.float32),
    )(e_lin, r_lin, g)


def kernel(x, row, alpha, W1, b1, W2, b2, ln_g, ln_b, Wa, ba):
    b1_eff = (b1 + alpha[0, 0] * W1[:, D]).reshape(1, D)
    w1t = W1[:, :D].T.astype(jnp.bfloat16)
    w2t = W2.T.astype(jnp.bfloat16)
    wa_pad = jnp.zeros((HP, D), jnp.float32).at[:4].set(Wa)
    wa_g = wa_pad * ln_g[None, :]
    wat = wa_g.T.astype(jnp.bfloat16)
    swa = jnp.sum(wa_g, axis=1).reshape(1, HP)
    bb = (jnp.zeros((HP,), jnp.float32).at[:4].set(ba)
          + wa_pad @ ln_b).reshape(1, HP)
    mask = jnp.zeros((1, HP), jnp.float32).at[0, :4].set(1.0)
    row3 = row.reshape(NW, NCH, CW)
    zeros = jnp.zeros((NSEGP, HP), jnp.float32)

    g = (jnp.arange(128)[:, None] // HP ==
         jnp.arange(8)[None, :]).astype(jnp.float32)

    e_lin = _tc1(x, w1t, w2t, wat, b1_eff, b2.reshape(1, D), swa, bb, mask)
    e16 = e_lin.reshape(N, HP)
    partials = _sc_k1(e16, row3, zeros)
    recip = _tcmid(partials)
    r16 = _sc_k2(recip, row3)
    out_lin = _tc2(e_lin, r16.reshape(LIN, 128), g)
    return out_lin.reshape(N, 1)


# speedup vs baseline: 7.0323x; 1.1656x over previous
import functools

import jax
import jax.numpy as jnp
from jax import lax
from jax.experimental import pallas as pl
from jax.experimental.pallas import tpu as pltpu
from jax.experimental.pallas import tpu_sc as plsc

N = 160000
D = 256
HP = 16
NSEG = 10000
EPS = 1e-5

TB = 1600
TB2L = 2048
LIN = N * HP // 128
OLIN = N // 128

NW = 32
CH = N // NW
CW = 125
NCH = CH // CW
NSEGP = 10240
SEG_STRIPE = NSEGP // 16


def _tc1_body(x_ref, w1_ref, w2_ref, wa_ref, b1_ref, b2_ref, swa_ref, bb_ref,
              mask_ref, o_ref):
    xb = x_ref[...].astype(jnp.bfloat16)
    h = jnp.dot(xb, w1_ref[...], preferred_element_type=jnp.float32)
    h = jnp.maximum(h + b1_ref[...], 0.0)
    h = jnp.dot(h.astype(jnp.bfloat16), w2_ref[...],
                preferred_element_type=jnp.float32)
    h = jnp.maximum(h + b2_ref[...], 0.0)
    s1 = jnp.sum(h, axis=-1, keepdims=True)
    s2 = jnp.sum(h * h, axis=-1, keepdims=True)
    mu = s1 * (1.0 / D)
    var = s2 * (1.0 / D) - mu * mu
    inv = jax.lax.rsqrt(var + EPS)
    t = jnp.dot(h.astype(jnp.bfloat16), wa_ref[...],
                preferred_element_type=jnp.float32)
    raw = inv * (t - mu * swa_ref[...]) + bb_ref[...]
    s = jax.nn.sigmoid(raw)
    e = jnp.exp(s) * mask_ref[...]
    e3 = e.reshape(TB // 8, 8, HP)
    o_ref[...] = jnp.concatenate([e3[:, u, :] for u in range(8)], axis=1)


def _tc1(x, w1t, w2t, wat, b1r, b2r, swar, bbr, maskr):
    grid = (N // TB,)
    full = lambda shape: pl.BlockSpec(shape, lambda i: (0, 0))
    return pl.pallas_call(
        _tc1_body,
        grid=grid,
        in_specs=[
            pl.BlockSpec((TB, D), lambda i: (i, 0)),
            full((D, D)), full((D, D)), full((D, HP)),
            full((1, D)), full((1, D)),
            full((1, HP)), full((1, HP)), full((1, HP)),
        ],
        out_specs=pl.BlockSpec((TB * HP // 128, 128), lambda i: (i, 0)),
        out_shape=jax.ShapeDtypeStruct((LIN, 128), jnp.float32),
    )(x, w1t, w2t, wat, b1r, b2r, swar, bbr, maskr)


def _sc_mesh():
    return plsc.VectorSubcoreMesh(core_axis_name="c", subcore_axis_name="s")


@jax.jit
def _sc_k1(e, row3, zeros):
    @functools.partial(
        pl.kernel,
        out_type=jax.ShapeDtypeStruct((2, NSEGP, HP), jnp.float32),
        mesh=_sc_mesh(),
        compiler_params=pltpu.CompilerParams(use_tc_tiling_on_sc=False),
        scratch_types=[
            pltpu.VMEM((CH, HP), jnp.float32),
            pltpu.VMEM((NCH, CW), jnp.int32),
            pltpu.VMEM_SHARED((NSEGP, HP), jnp.float32),
            pltpu.SemaphoreType.DMA,
        ],
    )
    def k(e_hbm, row_hbm, z_hbm, p_hbm, e_v, row_v, acc_sh, sem):
        c = lax.axis_index("c")
        s = lax.axis_index("s")
        wid = s * 2 + c

        @pl.when(s == 0)
        def _():
            pltpu.sync_copy(z_hbm, acc_sh)

        pltpu.async_copy(e_hbm.at[pl.ds(wid * CH, CH)], e_v, sem).wait()
        pltpu.async_copy(row_hbm.at[wid], row_v, sem).wait()
        plsc.subcore_barrier()

        @pl.loop(0, NCH)
        def _(j):
            pltpu.sync_copy(e_v.at[pl.ds(j * CW, CW)],
                            acc_sh.at[row_v.at[j]], add=True)

        plsc.subcore_barrier()
        pltpu.sync_copy(acc_sh.at[pl.ds(s * SEG_STRIPE, SEG_STRIPE)],
                        p_hbm.at[c].at[pl.ds(s * SEG_STRIPE, SEG_STRIPE)])

    return k(e, row3, zeros)


@jax.jit
def _sc_k2(recip, row3):
    @functools.partial(
        pl.kernel,
        out_type=jax.ShapeDtypeStruct((N, HP), jnp.float32),
        mesh=_sc_mesh(),
        compiler_params=pltpu.CompilerParams(use_tc_tiling_on_sc=False),
        scratch_types=[
            pltpu.VMEM((CH, HP), jnp.float32),
            pltpu.VMEM((NCH, CW), jnp.int32),
            pltpu.VMEM_SHARED((NSEGP, HP), jnp.float32),
            pltpu.SemaphoreType.DMA,
        ],
    )
    def k(recip_hbm, row_hbm, r_hbm, g_v, row_v, recip_sh, sem):
        c = lax.axis_index("c")
        s = lax.axis_index("s")
        wid = s * 2 + c

        @pl.when(s == 0)
        def _():
            pltpu.sync_copy(recip_hbm, recip_sh)

        pltpu.async_copy(row_hbm.at[wid], row_v, sem).wait()
        plsc.subcore_barrier()

        @pl.loop(0, NCH)
        def _(j):
            pltpu.sync_copy(recip_sh.at[row_v.at[j]],
                            g_v.at[pl.ds(j * CW, CW)])

        pltpu.sync_copy(g_v, r_hbm.at[pl.ds(wid * CH, CH)])

    return k(recip, row3)


def _tcmid_body(p_ref, o_ref):
    acc = p_ref[0] + p_ref[1]
    rec = jnp.where(acc > 0.0, 1.0 / acc, 0.0)
    r3 = rec.reshape(NSEGP // 8, 8, HP)
    o_ref[...] = jnp.concatenate([r3[:, u, :] for u in range(8)], axis=1)


def _tcmid(partials):
    return pl.pallas_call(
        _tcmid_body,
        in_specs=[pl.BlockSpec((2, NSEGP, HP), lambda: (0, 0, 0))],
        out_specs=pl.BlockSpec((NSEGP * HP // 128, 128), lambda: (0, 0)),
        out_shape=jax.ShapeDtypeStruct((NSEGP * HP // 128, 128), jnp.float32),
    )(partials)


def _tc2_body(e_ref, r_ref, g_ref, o_ref):
    prod = e_ref[...] * r_ref[...]
    sums = 0.25 * jnp.dot(prod, g_ref[...], preferred_element_type=jnp.float32)
    s3 = sums.reshape(TB2L // 16, 16, 8)
    o_ref[...] = jnp.concatenate([s3[:, u, :] for u in range(16)], axis=1)


def _tc2(e_lin, r_lin, g):
    grid = ((LIN + TB2L - 1) // TB2L,)
    return pl.pallas_call(
        _tc2_body,
        grid=grid,
        in_specs=[pl.BlockSpec((TB2L, 128), lambda i: (i, 0)),
                  pl.BlockSpec((TB2L, 128), lambda i: (i, 0)),
                  pl.BlockSpec((128, 8), lambda i: (0, 0))],
        out_specs=pl.BlockSpec((TB2L // 16, 128), lambda i: (i, 0)),
        out_shape=jax.ShapeDtypeStruct((OLIN, 128), jnp.float32),
    )(e_lin, r_lin, g)


def kernel(x, row, alpha, W1, b1, W2, b2, ln_g, ln_b, Wa, ba):
    b1_eff = (b1 + alpha[0, 0] * W1[:, D]).reshape(1, D)
    w1t = W1[:, :D].T.astype(jnp.bfloat16)
    w2t = W2.T.astype(jnp.bfloat16)
    wa_pad = jnp.zeros((HP, D), jnp.float32).at[:4].set(Wa)
    wa_g = wa_pad * ln_g[None, :]
    wat = wa_g.T.astype(jnp.bfloat16)
    swa = jnp.sum(wa_g, axis=1).reshape(1, HP)
    bb = (jnp.zeros((HP,), jnp.float32).at[:4].set(ba)
          + wa_pad @ ln_b).reshape(1, HP)
    mask = jnp.zeros((1, HP), jnp.float32).at[0, :4].set(1.0)
    row3 = row.reshape(NW, NCH, CW)
    zeros = jnp.zeros((NSEGP, HP), jnp.float32)

    g = (jnp.arange(128)[:, None] // HP ==
         jnp.arange(8)[None, :]).astype(jnp.float32)

    e_lin = _tc1(x, w1t, w2t, wat, b1_eff, b2.reshape(1, D), swa, bb, mask)
    e16 = e_lin.reshape(N, HP)
    partials = _sc_k1(e16, row3, zeros)
    recip_lin = _tcmid(partials)
    r16 = _sc_k2(recip_lin.reshape(NSEGP, HP), row3)
    out_lin = _tc2(e_lin, r16.reshape(LIN, 128), g)
    return out_lin.reshape(N, 1)


# speedup vs baseline: 7.1289x; 1.0137x over previous
import functools

import jax
import jax.numpy as jnp
from jax import lax
from jax.experimental import pallas as pl
from jax.experimental.pallas import tpu as pltpu
from jax.experimental.pallas import tpu_sc as plsc

N = 160000
D = 256
NH = 4
HP = 8
NSEG = 10000
EPS = 1e-5

LIN = N * HP // 128
OLIN = N // 128

TB = 1280
TB2L = 1024

NW = 32
CH = N // NW
CW = 125
NCH = CH // CW
NSEGP = 10240
SEG_STRIPE = NSEGP // 16


def _pack_lanes(v, groups):
    r, k = v.shape
    v3 = v.reshape(r // groups, groups, k)
    return jnp.concatenate([v3[:, u, :] for u in range(groups)], axis=1)


def _tc1_body(x_ref, w1_ref, w2_ref, wa_ref, b1_ref, b2_ref, swa_ref, bb_ref,
              mask_ref, o_ref):
    xb = x_ref[...].astype(jnp.bfloat16)
    h = jnp.dot(xb, w1_ref[...], preferred_element_type=jnp.float32)
    h = jnp.maximum(h + b1_ref[...], 0.0)
    h = jnp.dot(h.astype(jnp.bfloat16), w2_ref[...],
                preferred_element_type=jnp.float32)
    h = jnp.maximum(h + b2_ref[...], 0.0)
    s1 = jnp.sum(h, axis=-1, keepdims=True)
    s2 = jnp.sum(h * h, axis=-1, keepdims=True)
    mu = s1 * (1.0 / D)
    var = s2 * (1.0 / D) - mu * mu
    inv = jax.lax.rsqrt(var + EPS)
    t = jnp.dot(h.astype(jnp.bfloat16), wa_ref[...],
                preferred_element_type=jnp.float32)
    raw = inv * (t - mu * swa_ref[...]) + bb_ref[...]
    e = jnp.exp(jax.nn.sigmoid(raw)) * mask_ref[...]
    o_ref[...] = _pack_lanes(e, 128 // HP)


def _tc1(x, w1t, w2t, wat, b1r, b2r, swar, bbr, maskr):
    grid = (N // TB,)
    full = lambda shape: pl.BlockSpec(shape, lambda i: (0, 0))
    return pl.pallas_call(
        _tc1_body,
        grid=grid,
        in_specs=[
            pl.BlockSpec((TB, D), lambda i: (i, 0)),
            full((D, D)), full((D, D)), full((D, HP)),
            full((1, D)), full((1, D)),
            full((1, HP)), full((1, HP)), full((1, HP)),
        ],
        out_specs=pl.BlockSpec((TB * HP // 128, 128), lambda i: (i, 0)),
        out_shape=jax.ShapeDtypeStruct((LIN, 128), jnp.float32),
    )(x, w1t, w2t, wat, b1r, b2r, swar, bbr, maskr)


def _sc_mesh():
    return plsc.VectorSubcoreMesh(core_axis_name="c", subcore_axis_name="s")


@jax.jit
def _sc_k1(e, row3, zeros):
    @functools.partial(
        pl.kernel,
        out_type=jax.ShapeDtypeStruct((2, NSEGP, HP), jnp.float32),
        mesh=_sc_mesh(),
        compiler_params=pltpu.CompilerParams(use_tc_tiling_on_sc=False),
        scratch_types=[
            pltpu.VMEM((CH, HP), jnp.float32),
            pltpu.VMEM((NCH, CW), jnp.int32),
            pltpu.VMEM_SHARED((NSEGP, HP), jnp.float32),
            pltpu.SemaphoreType.DMA,
        ],
    )
    def k(e_hbm, row_hbm, z_hbm, p_hbm, e_v, row_v, acc_sh, sem):
        c = lax.axis_index("c")
        s = lax.axis_index("s")
        wid = s * 2 + c

        @pl.when(s == 0)
        def _():
            pltpu.sync_copy(z_hbm, acc_sh)

        pltpu.async_copy(e_hbm.at[pl.ds(wid * CH, CH)], e_v, sem).wait()
        pltpu.async_copy(row_hbm.at[wid], row_v, sem).wait()
        plsc.subcore_barrier()

        @pl.loop(0, NCH)
        def _(j):
            pltpu.sync_copy(e_v.at[pl.ds(j * CW, CW)],
                            acc_sh.at[row_v.at[j]], add=True)

        plsc.subcore_barrier()
        pltpu.sync_copy(acc_sh.at[pl.ds(s * SEG_STRIPE, SEG_STRIPE)],
                        p_hbm.at[c].at[pl.ds(s * SEG_STRIPE, SEG_STRIPE)])

    return k(e, row3, zeros)


@jax.jit
def _sc_k2(recip, row3):
    @functools.partial(
        pl.kernel,
        out_type=jax.ShapeDtypeStruct((N, HP), jnp.float32),
        mesh=_sc_mesh(),
        compiler_params=pltpu.CompilerParams(use_tc_tiling_on_sc=False),
        scratch_types=[
            pltpu.VMEM((CH, HP), jnp.float32),
            pltpu.VMEM((NCH, CW), jnp.int32),
            pltpu.VMEM_SHARED((NSEGP, HP), jnp.float32),
            pltpu.SemaphoreType.DMA,
        ],
    )
    def k(recip_hbm, row_hbm, r_hbm, g_v, row_v, recip_sh, sem):
        c = lax.axis_index("c")
        s = lax.axis_index("s")
        wid = s * 2 + c

        @pl.when(s == 0)
        def _():
            pltpu.sync_copy(recip_hbm, recip_sh)

        pltpu.async_copy(row_hbm.at[wid], row_v, sem).wait()
        plsc.subcore_barrier()

        @pl.loop(0, NCH)
        def _(j):
            pltpu.sync_copy(recip_sh.at[row_v.at[j]],
                            g_v.at[pl.ds(j * CW, CW)])

        pltpu.sync_copy(g_v, r_hbm.at[pl.ds(wid * CH, CH)])

    return k(recip, row3)


def _tcmid_body(p_ref, o_ref):
    acc = p_ref[0] + p_ref[1]
    rec = jnp.where(acc > 0.0, 1.0 / acc, 0.0)
    o_ref[...] = _pack_lanes(rec, 128 // HP)


def _tcmid(partials):
    return pl.pallas_call(
        _tcmid_body,
        in_specs=[pl.BlockSpec((2, NSEGP, HP), lambda: (0, 0, 0))],
        out_specs=pl.BlockSpec((NSEGP * HP // 128, 128), lambda: (0, 0)),
        out_shape=jax.ShapeDtypeStruct((NSEGP * HP // 128, 128),
                                       jnp.float32),
    )(partials)


def _tc2_body(e_ref, r_ref, g_ref, o_ref):
    prod = e_ref[...] * r_ref[...]
    sums = 0.25 * jnp.dot(prod, g_ref[...],
                          preferred_element_type=jnp.float32)
    o_ref[...] = _pack_lanes(sums, HP)


def _tc2(e_lin, r_lin, g):
    grid = ((LIN + TB2L - 1) // TB2L,)
    return pl.pallas_call(
        _tc2_body,
        grid=grid,
        in_specs=[pl.BlockSpec((TB2L, 128), lambda i: (i, 0)),
                  pl.BlockSpec((TB2L, 128), lambda i: (i, 0)),
                  pl.BlockSpec((128, 128 // HP), lambda i: (0, 0))],
        out_specs=pl.BlockSpec((TB2L // HP, 128), lambda i: (i, 0)),
        out_shape=jax.ShapeDtypeStruct((OLIN, 128), jnp.float32),
    )(e_lin, r_lin, g)


def kernel(x, row, alpha, W1, b1, W2, b2, ln_g, ln_b, Wa, ba):
    b1_eff = (b1 + alpha[0, 0] * W1[:, D]).reshape(1, D)
    w1t = W1[:, :D].T.astype(jnp.bfloat16)
    w2t = W2.T.astype(jnp.bfloat16)
    wa_pad = jnp.zeros((HP, D), jnp.float32).at[:NH].set(Wa)
    wa_g = wa_pad * ln_g[None, :]
    wat = wa_g.T.astype(jnp.bfloat16)
    swa = jnp.sum(wa_g, axis=1).reshape(1, HP)
    bb = (jnp.zeros((HP,), jnp.float32).at[:NH].set(ba)
          + wa_pad @ ln_b).reshape(1, HP)
    mask = jnp.zeros((1, HP), jnp.float32).at[0, :NH].set(1.0)
    row3 = row.reshape(NW, NCH, CW)
    zeros = jnp.zeros((NSEGP, HP), jnp.float32)
    g = (jnp.arange(128)[:, None] // HP ==
         jnp.arange(128 // HP)[None, :]).astype(jnp.float32)

    e_lin = _tc1(x, w1t, w2t, wat, b1_eff, b2.reshape(1, D), swa, bb, mask)
    e8 = e_lin.reshape(N, HP)
    partials = _sc_k1(e8, row3, zeros)
    recip_lin = _tcmid(partials)
    r8 = _sc_k2(recip_lin.reshape(NSEGP, HP), row3)
    out_lin = _tc2(e_lin, r8.reshape(LIN, 128), g)
    return out_lin.reshape(N, 1)


# speedup vs baseline: 7.2822x; 1.0215x over previous
import functools

import jax
import jax.numpy as jnp
from jax import lax
from jax.experimental import pallas as pl
from jax.experimental.pallas import tpu as pltpu
from jax.experimental.pallas import tpu_sc as plsc

N = 160000
D = 256
NH = 4
HP = 8
NSEG = 10000
EPS = 1e-5

LIN = N * HP // 128
OLIN = N // 128

TB = 3200
TB2L = 1024

NW = 32
CH = N // NW
CW = 125
NCH = CH // CW
NSEGP = 10240
SEG_STRIPE = NSEGP // 16


def _pack_lanes(v, groups):
    r, k = v.shape
    v3 = v.reshape(r // groups, groups, k)
    return jnp.concatenate([v3[:, u, :] for u in range(groups)], axis=1)


def _tc1_body(x_ref, w1_ref, w2_ref, aug_ref, ones_ref, b1_ref, b2_ref,
              swa_ref, bb_ref, mask_ref, o_ref):
    xb = x_ref[...].astype(jnp.bfloat16)
    acc1 = jnp.dot(xb, w1_ref[...], preferred_element_type=jnp.float32)
    h1b = jnp.maximum(acc1.astype(jnp.bfloat16) + b1_ref[...], 0)
    acc2 = jnp.dot(h1b, w2_ref[...], preferred_element_type=jnp.float32)
    h2b = jnp.maximum(acc2.astype(jnp.bfloat16) + b2_ref[...], 0)
    sq = h2b * h2b
    t_aug = jnp.dot(h2b, aug_ref[...], preferred_element_type=jnp.float32)
    s2 = jnp.dot(sq, ones_ref[...], preferred_element_type=jnp.float32)
    t = t_aug[:, :HP]
    mu = t_aug[:, HP:HP + 1] * (1.0 / D)
    var = s2[:, 0:1] * (1.0 / D) - mu * mu
    inv = jax.lax.rsqrt(var + EPS)
    raw = inv * (t - mu * swa_ref[...]) + bb_ref[...]
    e = jnp.exp(jax.nn.sigmoid(raw)) * mask_ref[...]
    o_ref[...] = _pack_lanes(e, 128 // HP)


def _tc1(x, w1t, w2t, aug, ones8, b1r, b2r, swar, bbr, maskr):
    grid = (N // TB,)
    full = lambda shape: pl.BlockSpec(shape, lambda i: (0, 0))
    return pl.pallas_call(
        _tc1_body,
        grid=grid,
        in_specs=[
            pl.BlockSpec((TB, D), lambda i: (i, 0)),
            full((D, D)), full((D, D)), full((D, 2 * HP)), full((D, 8)),
            full((1, D)), full((1, D)),
            full((1, HP)), full((1, HP)), full((1, HP)),
        ],
        out_specs=pl.BlockSpec((TB * HP // 128, 128), lambda i: (i, 0)),
        out_shape=jax.ShapeDtypeStruct((LIN, 128), jnp.float32),
    )(x, w1t, w2t, aug, ones8, b1r, b2r, swar, bbr, maskr)


def _sc_mesh():
    return plsc.VectorSubcoreMesh(core_axis_name="c", subcore_axis_name="s")


@jax.jit
def _sc_k1(e, row3, zeros):
    @functools.partial(
        pl.kernel,
        out_type=jax.ShapeDtypeStruct((2, NSEGP, HP), jnp.float32),
        mesh=_sc_mesh(),
        compiler_params=pltpu.CompilerParams(use_tc_tiling_on_sc=False),
        scratch_types=[
            pltpu.VMEM((CH, HP), jnp.float32),
            pltpu.VMEM((NCH, CW), jnp.int32),
            pltpu.VMEM_SHARED((NSEGP, HP), jnp.float32),
            pltpu.SemaphoreType.DMA,
        ],
    )
    def k(e_hbm, row_hbm, z_hbm, p_hbm, e_v, row_v, acc_sh, sem):
        c = lax.axis_index("c")
        s = lax.axis_index("s")
        wid = s * 2 + c

        @pl.when(s == 0)
        def _():
            pltpu.sync_copy(z_hbm, acc_sh)

        pltpu.async_copy(e_hbm.at[pl.ds(wid * CH, CH)], e_v, sem).wait()
        pltpu.async_copy(row_hbm.at[wid], row_v, sem).wait()
        plsc.subcore_barrier()

        @pl.loop(0, NCH)
        def _(j):
            pltpu.sync_copy(e_v.at[pl.ds(j * CW, CW)],
                            acc_sh.at[row_v.at[j]], add=True)

        plsc.subcore_barrier()
        pltpu.sync_copy(acc_sh.at[pl.ds(s * SEG_STRIPE, SEG_STRIPE)],
                        p_hbm.at[c].at[pl.ds(s * SEG_STRIPE, SEG_STRIPE)])

    return k(e, row3, zeros)


@jax.jit
def _sc_k2(recip, row3):
    @functools.partial(
        pl.kernel,
        out_type=jax.ShapeDtypeStruct((N, HP), jnp.float32),
        mesh=_sc_mesh(),
        compiler_params=pltpu.CompilerParams(use_tc_tiling_on_sc=False),
        scratch_types=[
            pltpu.VMEM((CH, HP), jnp.float32),
            pltpu.VMEM((NCH, CW), jnp.int32),
            pltpu.VMEM_SHARED((NSEGP, HP), jnp.float32),
            pltpu.SemaphoreType.DMA,
        ],
    )
    def k(recip_hbm, row_hbm, r_hbm, g_v, row_v, recip_sh, sem):
        c = lax.axis_index("c")
        s = lax.axis_index("s")
        wid = s * 2 + c

        @pl.when(s == 0)
        def _():
            pltpu.sync_copy(recip_hbm, recip_sh)

        pltpu.async_copy(row_hbm.at[wid], row_v, sem).wait()
        plsc.subcore_barrier()

        @pl.loop(0, NCH)
        def _(j):
            pltpu.sync_copy(recip_sh.at[row_v.at[j]],
                            g_v.at[pl.ds(j * CW, CW)])

        pltpu.sync_copy(g_v, r_hbm.at[pl.ds(wid * CH, CH)])

    return k(recip, row3)


PHALF = NSEGP * HP // 128


def _tcmid_body(p_ref, o_ref):
    acc = p_ref[:PHALF, :] + p_ref[PHALF:, :]
    o_ref[...] = jnp.where(acc > 0.0, 1.0 / acc, 0.0)


def _tcmid(p_lin):
    return pl.pallas_call(
        _tcmid_body,
        in_specs=[pl.BlockSpec((2 * PHALF, 128), lambda: (0, 0))],
        out_specs=pl.BlockSpec((PHALF, 128), lambda: (0, 0)),
        out_shape=jax.ShapeDtypeStruct((PHALF, 128), jnp.float32),
    )(p_lin)


def _tc2_body(e_ref, r_ref, g_ref, o_ref):
    prod = e_ref[...] * r_ref[...]
    sums = 0.25 * jnp.dot(prod, g_ref[...],
                          preferred_element_type=jnp.float32)
    o_ref[...] = _pack_lanes(sums, HP)


def _tc2(e_lin, r_lin, g):
    grid = ((LIN + TB2L - 1) // TB2L,)
    return pl.pallas_call(
        _tc2_body,
        grid=grid,
        in_specs=[pl.BlockSpec((TB2L, 128), lambda i: (i, 0)),
                  pl.BlockSpec((TB2L, 128), lambda i: (i, 0)),
                  pl.BlockSpec((128, 128 // HP), lambda i: (0, 0))],
        out_specs=pl.BlockSpec((TB2L // HP, 128), lambda i: (i, 0)),
        out_shape=jax.ShapeDtypeStruct((OLIN, 128), jnp.float32),
    )(e_lin, r_lin, g)


def kernel(x, row, alpha, W1, b1, W2, b2, ln_g, ln_b, Wa, ba):
    b1_eff = (b1 + alpha[0, 0] * W1[:, D]).reshape(1, D).astype(jnp.bfloat16)
    w1t = W1[:, :D].T.astype(jnp.bfloat16)
    w2t = W2.T.astype(jnp.bfloat16)
    wa_pad = jnp.zeros((HP, D), jnp.float32).at[:NH].set(Wa)
    wa_g = wa_pad * ln_g[None, :]
    aug = (jnp.zeros((D, 2 * HP), jnp.float32)
           .at[:, :HP].set(wa_g.T).at[:, HP].set(1.0)).astype(jnp.bfloat16)
    ones8 = jnp.zeros((D, 8), jnp.float32).at[:, 0].set(1.0).astype(
        jnp.bfloat16)
    swa = jnp.sum(wa_g, axis=1).reshape(1, HP)
    bb = (jnp.zeros((HP,), jnp.float32).at[:NH].set(ba)
          + wa_pad @ ln_b).reshape(1, HP)
    mask = jnp.zeros((1, HP), jnp.float32).at[0, :NH].set(1.0)
    row3 = row.reshape(NW, NCH, CW)
    zeros = jnp.zeros((NSEGP, HP), jnp.float32)
    g = (jnp.arange(128)[:, None] // HP ==
         jnp.arange(128 // HP)[None, :]).astype(jnp.float32)

    e_lin = _tc1(x, w1t, w2t, aug, ones8, b1_eff,
                 b2.reshape(1, D).astype(jnp.bfloat16), swa, bb, mask)
    e8 = e_lin.reshape(N, HP)
    partials = _sc_k1(e8, row3, zeros)
    recip_lin = _tcmid(partials.reshape(2 * PHALF, 128))
    r8 = _sc_k2(recip_lin.reshape(NSEGP, HP), row3)
    out_lin = _tc2(e_lin, r8.reshape(LIN, 128), g)
    return out_lin.reshape(N, 1)
